# baseline (speedup 1.0000x reference)
import functools

import jax
import jax.numpy as jnp
from jax import lax
from jax.experimental import pallas as pl
from jax.experimental.pallas import tpu as pltpu

_LANES = 128
_RB = 4096


def _anfis_body(nvar, xrep_ref, aux_ref, rules_ref, orules_ref, out_ref,
                acc_ref):
    i = pl.program_id(0)
    nb = pl.num_programs(0)
    rb = rules_ref.shape[0]

    @pl.when(i == 0)
    def _init():
        acc_ref[...] = jnp.zeros_like(acc_ref)

    c = aux_ref[0:1, :]
    s = aux_ref[1:2, :]
    d = xrep_ref[...] - c
    fuzz = jnp.exp(-(d * d) / (2.0 * s * s))

    iota = lax.broadcasted_iota(jnp.int32, (rb, _LANES), 1)

    wmin = None
    for v in range(nvar):
        oh = (iota == rules_ref[:, v:v + 1]).astype(jnp.float32)
        wv = lax.dot_general(fuzz, oh, (((1,), (1,)), ((), ())),
                             preferred_element_type=jnp.float32)
        wmin = wv if wmin is None else jnp.minimum(wmin, wv)

    oc = aux_ref[2:3, :]
    oh0 = (iota == orules_ref[:, 0:1]).astype(jnp.float32)
    oh1 = (iota == orules_ref[:, 1:2]).astype(jnp.float32)
    ow0 = lax.dot_general(oh0, oc, (((1,), (1,)), ((), ())),
                          preferred_element_type=jnp.float32)
    ow1 = lax.dot_general(oh1, oc, (((1,), (1,)), ((), ())),
                          preferred_element_type=jnp.float32)
    ones = jnp.ones((rb, 1), jnp.float32)
    zeros = jnp.zeros((rb, 5), jnp.float32)
    owt = jnp.concatenate([ow0, ow1, ones, zeros], axis=1)

    acc_ref[...] += jnp.dot(wmin, owt, preferred_element_type=jnp.float32)

    @pl.when(i == nb - 1)
    def _finish():
        acc = acc_ref[...]
        den = jnp.maximum(acc[:, 2:3], 1e-12)
        out_ref[...] = acc[:, 0:2] / den


def kernel(x, centers, sigmas, out_centers, input_rules, output_rules):
    b, nvar = x.shape
    m = centers.shape[1]
    f = nvar * m
    r = input_rules.shape[0]
    nb = r // _RB

    xrep = jnp.broadcast_to(x[:, :, None], (b, nvar, m)).reshape(b, f)
    xrep = jnp.pad(xrep, ((0, 0), (0, _LANES - f)))
    aux = jnp.zeros((8, _LANES), jnp.float32)
    aux = aux.at[0, :f].set(centers.reshape(f))
    aux = aux.at[1, :].set(
        jnp.pad(sigmas.reshape(f), (0, _LANES - f), constant_values=1.0))
    aux = aux.at[2, :out_centers.shape[0]].set(out_centers)

    return pl.pallas_call(
        functools.partial(_anfis_body, nvar),
        grid=(nb,),
        in_specs=[
            pl.BlockSpec((b, _LANES), lambda i: (0, 0)),
            pl.BlockSpec((8, _LANES), lambda i: (0, 0)),
            pl.BlockSpec((_RB, nvar), lambda i: (i, 0)),
            pl.BlockSpec((_RB, 2), lambda i: (i, 0)),
        ],
        out_specs=pl.BlockSpec((b, 2), lambda i: (0, 0)),
        out_shape=jax.ShapeDtypeStruct((b, 2), jnp.float32),
        scratch_shapes=[pltpu.VMEM((b, 8), jnp.float32)],
    )(xrep, aux, input_rules, output_rules)


# speedup vs baseline: 1.3509x; 1.3509x over previous
import functools

import jax
import jax.numpy as jnp
from jax import lax
from jax.experimental import pallas as pl
from jax.experimental.pallas import tpu as pltpu

_LANES = 128
_RB = 8192


def _anfis_body(nvar, xrep_ref, aux_ref, rules_ref, orules_ref, out_ref,
                acc_ref):
    i = pl.program_id(0)
    nb = pl.num_programs(0)
    rb = rules_ref.shape[1]

    @pl.when(i == 0)
    def _init():
        acc_ref[...] = jnp.zeros_like(acc_ref)

    c = aux_ref[0:1, :]
    s = aux_ref[1:2, :]
    d = xrep_ref[...] - c
    fuzz = jnp.exp(-(d * d) / (2.0 * s * s))

    iota = lax.broadcasted_iota(jnp.int32, (_LANES, rb), 0)

    wmin = None
    for v in range(nvar):
        oh = (iota == rules_ref[v:v + 1, :]).astype(jnp.float32)
        wv = jnp.dot(fuzz, oh, preferred_element_type=jnp.float32)
        wmin = wv if wmin is None else jnp.minimum(wmin, wv)

    oc = aux_ref[...]
    oh0 = (iota == orules_ref[0:1, :]).astype(jnp.float32)
    oh1 = (iota == orules_ref[1:2, :]).astype(jnp.float32)
    row0 = jnp.dot(oc, oh0, preferred_element_type=jnp.float32)[2:3, :]
    row1 = jnp.dot(oc, oh1, preferred_element_type=jnp.float32)[2:3, :]
    ones = jnp.ones((1, rb), jnp.float32)
    zeros = jnp.zeros((5, rb), jnp.float32)
    owt = jnp.concatenate([row0, row1, ones, zeros], axis=0)

    acc_ref[...] += lax.dot_general(
        wmin, owt, (((1,), (1,)), ((), ())),
        preferred_element_type=jnp.float32)

    @pl.when(i == nb - 1)
    def _finish():
        acc = acc_ref[...]
        den = jnp.maximum(acc[:, 2:3], 1e-12)
        out_ref[...] = acc[:, 0:2] / den


def kernel(x, centers, sigmas, out_centers, input_rules, output_rules):
    b, nvar = x.shape
    m = centers.shape[1]
    f = nvar * m
    r = input_rules.shape[0]
    nb = r // _RB

    xrep = jnp.broadcast_to(x[:, :, None], (b, nvar, m)).reshape(b, f)
    xrep = jnp.pad(xrep, ((0, 0), (0, _LANES - f)))
    aux = jnp.zeros((8, _LANES), jnp.float32)
    aux = aux.at[0, :f].set(centers.reshape(f))
    aux = aux.at[1, :].set(
        jnp.pad(sigmas.reshape(f), (0, _LANES - f), constant_values=1.0))
    aux = aux.at[2, :out_centers.shape[0]].set(out_centers)
    rules_p = jnp.zeros((8, r), jnp.int32).at[:nvar, :].set(input_rules.T)
    orules_p = jnp.zeros((8, r), jnp.int32).at[:2, :].set(output_rules.T)

    return pl.pallas_call(
        functools.partial(_anfis_body, nvar),
        grid=(nb,),
        in_specs=[
            pl.BlockSpec((b, _LANES), lambda i: (0, 0)),
            pl.BlockSpec((8, _LANES), lambda i: (0, 0)),
            pl.BlockSpec((8, _RB), lambda i: (0, i)),
            pl.BlockSpec((8, _RB), lambda i: (0, i)),
        ],
        out_specs=pl.BlockSpec((b, 2), lambda i: (0, 0)),
        out_shape=jax.ShapeDtypeStruct((b, 2), jnp.float32),
        scratch_shapes=[pltpu.VMEM((b, 8), jnp.float32)],
    )(xrep, aux, rules_p, orules_p)
